# baseline (speedup 1.0000x reference)
import functools

import jax
import jax.numpy as jnp
from jax import lax
from jax.experimental import pallas as pl
from jax.experimental.pallas import tpu as pltpu
from jax.experimental.pallas import tpu_sc as plsc

_IDX = 13


@functools.lru_cache(maxsize=None)
def _make_sc_gather(V4, B):
    info = plsc.get_sparse_core_info()
    NC, NS = info.num_cores, info.num_subcores
    NW = NC * NS
    CH = 128
    b_per_w = B // NW
    n_ch = b_per_w // CH
    assert b_per_w % CH == 0

    mesh = plsc.VectorSubcoreMesh(core_axis_name="c", subcore_axis_name="s")

    @functools.partial(
        pl.kernel,
        mesh=mesh,
        out_type=jax.ShapeDtypeStruct((B, 128), jnp.float32),
        scratch_types=[
            pltpu.VMEM((n_ch, CH), jnp.int32),
            pltpu.VMEM((b_per_w, 128), jnp.float32),
            pltpu.SemaphoreType.DMA,
        ],
    )
    def gather_k(table_hbm, idx_hbm, out_hbm, idx_v, rows_v, sem):
        wid = lax.axis_index("s") * NC + lax.axis_index("c")
        pltpu.sync_copy(idx_hbm.at[pl.ds(wid * n_ch, n_ch)], idx_v)
        copies = [
            pltpu.async_copy(
                table_hbm.at[idx_v.at[j]],
                rows_v.at[pl.ds(j * CH, CH)],
                sem,
            )
            for j in range(n_ch)
        ]
        for c in copies:
            c.wait()
        pltpu.sync_copy(rows_v, out_hbm.at[pl.ds(wid * b_per_w, b_per_w)])

    return gather_k


def _mlp_body(x_ref, e4_ref, m_ref, w1d_ref, w1e_ref, b1_ref, w2_ref, b2_ref, o_ref):
    e4 = e4_ref[...]
    m = m_ref[...]
    emb = e4[:, 0:32]
    for r in range(1, 4):
        emb = jnp.where(m == r, e4[:, 32 * r : 32 * (r + 1)], emb)
    h = jnp.dot(x_ref[...], w1d_ref[...], preferred_element_type=jnp.float32)
    h = h + jnp.dot(emb, w1e_ref[...], preferred_element_type=jnp.float32)
    h = jnp.maximum(h + b1_ref[...], 0.0)
    o = jnp.dot(h, w2_ref[...], preferred_element_type=jnp.float32) + b2_ref[...]
    o_ref[...] = jnp.maximum(o, 0.0)


def _mlp(x, e4, m, W1d, W1e, b1, W2, b2, block_b=2048):
    B, F = x.shape
    HID = W2.shape[0]
    OUT = W2.shape[1]
    grid = (B // block_b,)
    return pl.pallas_call(
        _mlp_body,
        grid=grid,
        in_specs=[
            pl.BlockSpec((block_b, F), lambda i: (i, 0)),
            pl.BlockSpec((block_b, 128), lambda i: (i, 0)),
            pl.BlockSpec((block_b, 1), lambda i: (i, 0)),
            pl.BlockSpec((F, HID), lambda i: (0, 0)),
            pl.BlockSpec((32, HID), lambda i: (0, 0)),
            pl.BlockSpec((1, HID), lambda i: (0, 0)),
            pl.BlockSpec((HID, OUT), lambda i: (0, 0)),
            pl.BlockSpec((1, OUT), lambda i: (0, 0)),
        ],
        out_specs=pl.BlockSpec((block_b, OUT), lambda i: (i, 0)),
        out_shape=jax.ShapeDtypeStruct((B, OUT), jnp.float32),
    )(x, e4, m, W1d, W1e, b1, W2, b2)


def kernel(inputs, table, W1, b1, W2, b2):
    B, F = inputs.shape
    V, D = table.shape
    HID = W1.shape[1]
    idx = inputs[:, _IDX].astype(jnp.int32)
    table128 = table.reshape(V * D // 128, 128)
    idx4 = (idx // 4).reshape(-1, 128)
    m = (idx % 4).reshape(-1, 1)
    e4 = _make_sc_gather(V * D // 128, B)(table128, idx4)
    W1d = jnp.concatenate(
        [W1[:_IDX], jnp.zeros((1, HID), W1.dtype), W1[_IDX : F - 1]], axis=0
    )
    W1e = W1[F - 1 :]
    return _mlp(inputs, e4, m, W1d, W1e, b1.reshape(1, -1), W2, b2.reshape(1, -1))


# speedup vs baseline: 3.7970x; 3.7970x over previous
import functools

import jax
import jax.numpy as jnp
from jax import lax
from jax.experimental import pallas as pl
from jax.experimental.pallas import tpu as pltpu
from jax.experimental.pallas import tpu_sc as plsc

_IDX = 13
_LANE = 128
_GRP = 4
_NBUF = 3


@functools.lru_cache(maxsize=None)
def _make_sc_gather(V, D, B):
    info = plsc.get_sparse_core_info()
    NC, NS = info.num_cores, info.num_subcores
    NW = NC * NS
    b_per_w = B // NW
    n_chunks = b_per_w // 16
    n_grp = b_per_w // _GRP

    mesh = plsc.VectorSubcoreMesh(core_axis_name="c", subcore_axis_name="s")

    @functools.partial(
        pl.kernel,
        mesh=mesh,
        compiler_params=pltpu.CompilerParams(needs_layout_passes=False),
        out_type=jax.ShapeDtypeStruct((B, D), jnp.float32),
        scratch_types=[
            pltpu.VMEM((b_per_w,), jnp.int32),
            pltpu.VMEM((_NBUF * _GRP, D, _LANE), jnp.float32),
            pltpu.VMEM((b_per_w, D), jnp.float32),
            pltpu.SemaphoreType.DMA,
        ],
    )
    def gather_k(tableT_hbm, idx_hbm, out_hbm, idx_v, slabs_v, rows_v, sem):
        wid = lax.axis_index("s") * NC + lax.axis_index("c")
        base = wid * b_per_w
        pltpu.sync_copy(idx_hbm.at[pl.ds(base, b_per_w)], idx_v)
        iota16 = lax.iota(jnp.int32, 16)

        def fire(g, chunk, lane0):
            slot = (g % _NBUF) * _GRP
            for i in range(_GRP):
                sid = chunk[lane0 + i]
                colbase = pl.multiple_of((sid // _LANE) * _LANE, _LANE)
                pltpu.async_copy(
                    tableT_hbm.at[:, pl.ds(colbase, _LANE)],
                    slabs_v.at[slot + i],
                    sem,
                )

        def process(g, chunk, lane0):
            slot = (g % _NBUF) * _GRP
            for _ in range(_GRP):
                pltpu.make_async_copy(
                    tableT_hbm.at[:, pl.ds(0, _LANE)],
                    slabs_v.at[0],
                    sem,
                ).wait()
            row0 = g * _GRP
            for i in range(_GRP):
                sid = chunk[lane0 + i]
                col = jnp.full((16,), sid % _LANE, jnp.int32)
                rowv = jnp.full((16,), row0 + i, jnp.int32)
                slotv = jnp.full((16,), slot + i, jnp.int32)
                for h in range(D // 16):
                    vals = plsc.load_gather(
                        slabs_v, [slotv, iota16 + 16 * h, col]
                    )
                    plsc.store_scatter(
                        rows_v, [rowv, iota16 + 16 * h], vals
                    )

        c0 = idx_v[pl.ds(0, 16)]
        fire(0, c0, 0)
        fire(1, c0, 4)

        def body(c, carry):
            cur = idx_v[pl.ds(c * 16, 16)]
            nxt = idx_v[pl.ds((c + 1) * 16, 16)]
            for sub in range(4):
                g = c * 4 + sub
                if True:
                    fsub = sub + 2
                    if fsub < 4:
                        fire(g + 2, cur, 4 * fsub)
                    else:
                        fire(g + 2, nxt, 4 * (fsub - 4))
                process(g, cur, 4 * sub)
            return carry

        lax.fori_loop(0, n_chunks - 1, body, 0)
        clast = idx_v[pl.ds((n_chunks - 1) * 16, 16)]
        for sub in range(4):
            g = (n_chunks - 1) * 4 + sub
            if sub + 2 < 4:
                fire(g + 2, clast, 4 * (sub + 2))
            process(g, clast, 4 * sub)

        pltpu.sync_copy(rows_v, out_hbm.at[pl.ds(base, b_per_w)])

    return gather_k


def _mlp_body(x_ref, e_ref, w1d_ref, w1e_ref, b1_ref, w2_ref, b2_ref, o_ref):
    h = jnp.dot(x_ref[...], w1d_ref[...], preferred_element_type=jnp.float32)
    h = h + jnp.dot(e_ref[...], w1e_ref[...], preferred_element_type=jnp.float32)
    h = jnp.maximum(h + b1_ref[...], 0.0)
    o = jnp.dot(h, w2_ref[...], preferred_element_type=jnp.float32) + b2_ref[...]
    o_ref[...] = jnp.maximum(o, 0.0)


def _mlp(x, emb, W1d, W1e, b1, W2, b2, block_b=2048):
    B, F = x.shape
    HID = W2.shape[0]
    OUT = W2.shape[1]
    D = emb.shape[1]
    grid = (B // block_b,)
    return pl.pallas_call(
        _mlp_body,
        grid=grid,
        in_specs=[
            pl.BlockSpec((block_b, F), lambda i: (i, 0)),
            pl.BlockSpec((block_b, D), lambda i: (i, 0)),
            pl.BlockSpec((F, HID), lambda i: (0, 0)),
            pl.BlockSpec((D, HID), lambda i: (0, 0)),
            pl.BlockSpec((1, HID), lambda i: (0, 0)),
            pl.BlockSpec((HID, OUT), lambda i: (0, 0)),
            pl.BlockSpec((1, OUT), lambda i: (0, 0)),
        ],
        out_specs=pl.BlockSpec((block_b, OUT), lambda i: (i, 0)),
        out_shape=jax.ShapeDtypeStruct((B, OUT), jnp.float32),
    )(x, emb, W1d, W1e, b1, W2, b2)


def kernel(inputs, table, W1, b1, W2, b2):
    B, F = inputs.shape
    V, D = table.shape
    HID = W1.shape[1]
    tableT = table.T
    idx = inputs[:, _IDX].astype(jnp.int32)
    W1d = jnp.concatenate(
        [W1[:_IDX], jnp.zeros((1, HID), W1.dtype), W1[_IDX : F - 1]], axis=0
    )
    W1e = W1[F - 1 :]
    b1r = b1.reshape(1, -1)
    b2r = b2.reshape(1, -1)
    H = B // 2
    gat = _make_sc_gather(V, D, H)
    outs = []
    for h in range(2):
        emb_h = gat(tableT, idx[h * H : (h + 1) * H])
        outs.append(
            _mlp(inputs[h * H : (h + 1) * H], emb_h, W1d, W1e, b1r, W2, b2r)
        )
    return jnp.concatenate(outs, axis=0)


# speedup vs baseline: 4.5526x; 1.1990x over previous
import functools

import jax
import jax.numpy as jnp
from jax import lax
from jax.experimental import pallas as pl
from jax.experimental.pallas import tpu as pltpu
from jax.experimental.pallas import tpu_sc as plsc

_IDX = 13
_LANE = 128
_GRP = 4
_NBUF = 3


@functools.lru_cache(maxsize=None)
def _make_sc_gather(V, D, B):
    info = plsc.get_sparse_core_info()
    NC, NS = info.num_cores, info.num_subcores
    NW = NC * NS
    b_per_w = B // NW
    n_chunks = b_per_w // 16
    n_grp = b_per_w // _GRP

    mesh = plsc.VectorSubcoreMesh(core_axis_name="c", subcore_axis_name="s")

    @functools.partial(
        pl.kernel,
        mesh=mesh,
        compiler_params=pltpu.CompilerParams(needs_layout_passes=False),
        out_type=jax.ShapeDtypeStruct((D, B), jnp.float32),
        scratch_types=[
            pltpu.VMEM((b_per_w,), jnp.int32),
            pltpu.VMEM((_NBUF * _GRP, D, _LANE), jnp.float32),
            pltpu.VMEM((D, b_per_w), jnp.float32),
            pltpu.SemaphoreType.DMA,
        ],
    )
    def gather_k(tableT_hbm, idx_hbm, out_hbm, idx_v, slabs_v, rows_v, sem):
        wid = lax.axis_index("s") * NC + lax.axis_index("c")
        base = wid * b_per_w
        pltpu.sync_copy(idx_hbm.at[pl.ds(base, b_per_w)], idx_v)
        iota16 = lax.iota(jnp.int32, 16)

        def fire(g, chunk, lane0):
            slot = (g % _NBUF) * _GRP
            for i in range(_GRP):
                sid = chunk[lane0 + i]
                colbase = pl.multiple_of((sid // _LANE) * _LANE, _LANE)
                pltpu.async_copy(
                    tableT_hbm.at[:, pl.ds(colbase, _LANE)],
                    slabs_v.at[slot + i],
                    sem,
                )

        def process(g, chunk, lane0):
            slot = (g % _NBUF) * _GRP
            for _ in range(_GRP):
                pltpu.make_async_copy(
                    tableT_hbm.at[:, pl.ds(0, _LANE)],
                    slabs_v.at[0],
                    sem,
                ).wait()
            row0 = g * _GRP
            for i in range(_GRP):
                sid = chunk[lane0 + i]
                col = jnp.full((16,), sid % _LANE, jnp.int32)
                posv = jnp.full((16,), row0 + i, jnp.int32)
                slotv = jnp.full((16,), slot + i, jnp.int32)
                for h in range(D // 16):
                    vals = plsc.load_gather(
                        slabs_v, [slotv, iota16 + 16 * h, col]
                    )
                    plsc.store_scatter(
                        rows_v, [iota16 + 16 * h, posv], vals
                    )

        c0 = idx_v[pl.ds(0, 16)]
        fire(0, c0, 0)
        fire(1, c0, 4)

        def body(c, carry):
            cur = idx_v[pl.ds(c * 16, 16)]
            nxt = idx_v[pl.ds((c + 1) * 16, 16)]
            for sub in range(4):
                g = c * 4 + sub
                if True:
                    fsub = sub + 2
                    if fsub < 4:
                        fire(g + 2, cur, 4 * fsub)
                    else:
                        fire(g + 2, nxt, 4 * (fsub - 4))
                process(g, cur, 4 * sub)
            return carry

        lax.fori_loop(0, n_chunks - 1, body, 0)
        clast = idx_v[pl.ds((n_chunks - 1) * 16, 16)]
        for sub in range(4):
            g = (n_chunks - 1) * 4 + sub
            if sub + 2 < 4:
                fire(g + 2, clast, 4 * (sub + 2))
            process(g, clast, 4 * sub)

        pltpu.sync_copy(rows_v, out_hbm.at[:, pl.ds(base, b_per_w)])

    return gather_k


def _mlp_body(xT_ref, eT_ref, w1dT_ref, w1eT_ref, b1_ref, w2T_ref, b2_ref, oT_ref):
    h = jnp.dot(w1dT_ref[...], xT_ref[...], preferred_element_type=jnp.float32)
    h = h + jnp.dot(w1eT_ref[...], eT_ref[...], preferred_element_type=jnp.float32)
    h = jnp.maximum(h + b1_ref[...], 0.0)
    o = jnp.dot(w2T_ref[...], h, preferred_element_type=jnp.float32) + b2_ref[...]
    oT_ref[...] = jnp.maximum(o, 0.0)


def _mlp(xT, embT, W1dT, W1eT, b1c, W2T, b2c, block_b=2048):
    F, B = xT.shape
    OUT, HID = W2T.shape
    D = embT.shape[0]
    grid = (B // block_b,)
    return pl.pallas_call(
        _mlp_body,
        grid=grid,
        in_specs=[
            pl.BlockSpec((F, block_b), lambda i: (0, i)),
            pl.BlockSpec((D, block_b), lambda i: (0, i)),
            pl.BlockSpec((HID, F), lambda i: (0, 0)),
            pl.BlockSpec((HID, D), lambda i: (0, 0)),
            pl.BlockSpec((HID, 1), lambda i: (0, 0)),
            pl.BlockSpec((OUT, HID), lambda i: (0, 0)),
            pl.BlockSpec((OUT, 1), lambda i: (0, 0)),
        ],
        out_specs=pl.BlockSpec((OUT, block_b), lambda i: (0, i)),
        out_shape=jax.ShapeDtypeStruct((OUT, B), jnp.float32),
    )(xT, embT, W1dT, W1eT, b1c, W2T, b2c)


def kernel(inputs, table, W1, b1, W2, b2):
    B, F = inputs.shape
    V, D = table.shape
    HID = W1.shape[1]
    tableT = table.T
    inputsT = inputs.T
    idx = inputsT[_IDX].astype(jnp.int32)
    embT = _make_sc_gather(V, D, B)(tableT, idx)
    W1T = W1.T
    W1dT = jnp.concatenate(
        [W1T[:, :_IDX], jnp.zeros((HID, 1), W1.dtype), W1T[:, _IDX : F - 1]],
        axis=1,
    )
    W1eT = W1T[:, F - 1 :]
    outT = _mlp(
        inputsT, embT, W1dT, W1eT, b1.reshape(-1, 1), W2.T, b2.reshape(-1, 1)
    )
    return outT.T


# speedup vs baseline: 4.6422x; 1.0197x over previous
import functools

import jax
import jax.numpy as jnp
from jax import lax
from jax.experimental import pallas as pl
from jax.experimental.pallas import tpu as pltpu
from jax.experimental.pallas import tpu_sc as plsc

_IDX = 13
_LANE = 128
_GRP = 4
_NBUF = 3


@functools.lru_cache(maxsize=None)
def _make_sc_gather(V, D, B):
    info = plsc.get_sparse_core_info()
    NC, NS = info.num_cores, info.num_subcores
    NW = NC * NS
    b_per_w = B // NW
    n_chunks = b_per_w // 16
    n_grp = b_per_w // _GRP

    mesh = plsc.VectorSubcoreMesh(core_axis_name="c", subcore_axis_name="s")

    @functools.partial(
        pl.kernel,
        mesh=mesh,
        compiler_params=pltpu.CompilerParams(needs_layout_passes=False),
        out_type=jax.ShapeDtypeStruct((D, B), jnp.float32),
        scratch_types=[
            pltpu.VMEM((b_per_w,), jnp.int32),
            pltpu.VMEM((_NBUF * _GRP, D, _LANE), jnp.float32),
            pltpu.VMEM((D, b_per_w), jnp.float32),
            pltpu.SemaphoreType.DMA,
        ],
    )
    def gather_k(tableT_hbm, idx_hbm, out_hbm, idx_v, slabs_v, rows_v, sem):
        wid = lax.axis_index("s") * NC + lax.axis_index("c")
        base = wid * b_per_w
        pltpu.sync_copy(idx_hbm.at[pl.ds(base, b_per_w)], idx_v)
        iota16 = lax.iota(jnp.int32, 16)

        def fire(g, chunk, lane0):
            slot = (g % _NBUF) * _GRP
            for i in range(_GRP):
                sid = chunk[lane0 + i]
                colbase = pl.multiple_of((sid // _LANE) * _LANE, _LANE)
                pltpu.async_copy(
                    tableT_hbm.at[:, pl.ds(colbase, _LANE)],
                    slabs_v.at[slot + i],
                    sem,
                )

        def process(g, chunk, lane0):
            slot = (g % _NBUF) * _GRP
            for _ in range(_GRP):
                pltpu.make_async_copy(
                    tableT_hbm.at[:, pl.ds(0, _LANE)],
                    slabs_v.at[0],
                    sem,
                ).wait()
            row0 = g * _GRP
            for i in range(_GRP):
                sid = chunk[lane0 + i]
                col = jnp.full((16,), sid % _LANE, jnp.int32)
                posv = jnp.full((16,), row0 + i, jnp.int32)
                slotv = jnp.full((16,), slot + i, jnp.int32)
                for h in range(D // 16):
                    vals = plsc.load_gather(
                        slabs_v, [slotv, iota16 + 16 * h, col]
                    )
                    plsc.store_scatter(
                        rows_v, [iota16 + 16 * h, posv], vals
                    )

        c0 = idx_v[pl.ds(0, 16)]
        fire(0, c0, 0)
        fire(1, c0, 4)

        def body(c, carry):
            cur = idx_v[pl.ds(c * 16, 16)]
            nxt = idx_v[pl.ds((c + 1) * 16, 16)]
            for sub in range(4):
                g = c * 4 + sub
                if True:
                    fsub = sub + 2
                    if fsub < 4:
                        fire(g + 2, cur, 4 * fsub)
                    else:
                        fire(g + 2, nxt, 4 * (fsub - 4))
                process(g, cur, 4 * sub)
            return carry

        lax.fori_loop(0, n_chunks - 1, body, 0)
        clast = idx_v[pl.ds((n_chunks - 1) * 16, 16)]
        for sub in range(4):
            g = (n_chunks - 1) * 4 + sub
            if sub + 2 < 4:
                fire(g + 2, clast, 4 * (sub + 2))
            process(g, clast, 4 * sub)

        pltpu.sync_copy(rows_v, out_hbm.at[:, pl.ds(base, b_per_w)])

    return gather_k


def _mlp_body(xT_ref, eT_ref, w1dT_ref, w1eT_ref, b1_ref, w2T_ref, b2_ref, oT_ref):
    h = jnp.dot(w1dT_ref[...], xT_ref[...], preferred_element_type=jnp.float32)
    h = h + jnp.dot(w1eT_ref[...], eT_ref[...], preferred_element_type=jnp.float32)
    h = jnp.maximum(h + b1_ref[...], 0.0)
    o = jnp.dot(w2T_ref[...], h, preferred_element_type=jnp.float32) + b2_ref[...]
    oT_ref[...] = jnp.maximum(o, 0.0)


def _mlp(xT, embT, W1dT, W1eT, b1c, W2T, b2c, block_b=4096):
    F, B = xT.shape
    OUT, HID = W2T.shape
    D = embT.shape[0]
    grid = (B // block_b,)
    return pl.pallas_call(
        _mlp_body,
        grid=grid,
        in_specs=[
            pl.BlockSpec((F, block_b), lambda i: (0, i)),
            pl.BlockSpec((D, block_b), lambda i: (0, i)),
            pl.BlockSpec((HID, F), lambda i: (0, 0)),
            pl.BlockSpec((HID, D), lambda i: (0, 0)),
            pl.BlockSpec((HID, 1), lambda i: (0, 0)),
            pl.BlockSpec((OUT, HID), lambda i: (0, 0)),
            pl.BlockSpec((OUT, 1), lambda i: (0, 0)),
        ],
        out_specs=pl.BlockSpec((OUT, block_b), lambda i: (0, i)),
        out_shape=jax.ShapeDtypeStruct((OUT, B), jnp.float32),
    )(xT, embT, W1dT, W1eT, b1c, W2T, b2c)


def kernel(inputs, table, W1, b1, W2, b2):
    B, F = inputs.shape
    V, D = table.shape
    HID = W1.shape[1]
    tableT = table.T
    inputsT = inputs.T
    idx = inputsT[_IDX].astype(jnp.int32)
    embT = _make_sc_gather(V, D, B)(tableT, idx)
    W1T = W1.T
    W1dT = jnp.concatenate(
        [W1T[:, :_IDX], jnp.zeros((HID, 1), W1.dtype), W1T[:, _IDX : F - 1]],
        axis=1,
    )
    W1eT = W1T[:, F - 1 :]
    outT = _mlp(
        inputsT, embT, W1dT, W1eT, b1.reshape(-1, 1), W2.T, b2.reshape(-1, 1)
    )
    return outT.T


# speedup vs baseline: 4.6716x; 1.0063x over previous
import functools

import jax
import jax.numpy as jnp
from jax import lax
from jax.experimental import pallas as pl
from jax.experimental.pallas import tpu as pltpu
from jax.experimental.pallas import tpu_sc as plsc

_IDX = 13
_LANE = 128
_GRP = 4
_NBUF = 3


@functools.lru_cache(maxsize=None)
def _make_sc_gather(V, D, B):
    info = plsc.get_sparse_core_info()
    NC, NS = info.num_cores, info.num_subcores
    NW = NC * NS
    b_per_w = B // NW
    n_chunks = b_per_w // 16
    n_grp = b_per_w // _GRP

    mesh = plsc.VectorSubcoreMesh(core_axis_name="c", subcore_axis_name="s")

    @functools.partial(
        pl.kernel,
        mesh=mesh,
        compiler_params=pltpu.CompilerParams(needs_layout_passes=False),
        out_type=jax.ShapeDtypeStruct((D, B), jnp.float32),
        scratch_types=[
            pltpu.VMEM((b_per_w,), jnp.int32),
            pltpu.VMEM((_NBUF * _GRP, D, _LANE), jnp.float32),
            pltpu.VMEM((D, b_per_w), jnp.float32),
            pltpu.SemaphoreType.DMA,
        ],
    )
    def gather_k(tableT_hbm, idx_hbm, out_hbm, idx_v, slabs_v, rows_v, sem):
        wid = lax.axis_index("s") * NC + lax.axis_index("c")
        base = wid * b_per_w
        pltpu.sync_copy(idx_hbm.at[pl.ds(base, b_per_w)], idx_v)
        iota16 = lax.iota(jnp.int32, 16)

        def fire(g, chunk, lane0):
            slot = (g % _NBUF) * _GRP
            for i in range(_GRP):
                sid = chunk[lane0 + i]
                colbase = pl.multiple_of((sid // _LANE) * _LANE, _LANE)
                pltpu.async_copy(
                    tableT_hbm.at[:, pl.ds(colbase, _LANE)],
                    slabs_v.at[slot + i],
                    sem,
                )

        def process(g, chunk, lane0):
            slot = (g % _NBUF) * _GRP
            for _ in range(_GRP):
                pltpu.make_async_copy(
                    tableT_hbm.at[:, pl.ds(0, _LANE)],
                    slabs_v.at[0],
                    sem,
                ).wait()
            row0 = g * _GRP
            for i in range(_GRP):
                sid = chunk[lane0 + i]
                col = jnp.full((16,), sid % _LANE, jnp.int32)
                posv = jnp.full((16,), row0 + i, jnp.int32)
                slotv = jnp.full((16,), slot + i, jnp.int32)
                for h in range(D // 16):
                    vals = plsc.load_gather(
                        slabs_v, [slotv, iota16 + 16 * h, col]
                    )
                    plsc.store_scatter(
                        rows_v, [iota16 + 16 * h, posv], vals
                    )

        c0 = idx_v[pl.ds(0, 16)]
        fire(0, c0, 0)
        fire(1, c0, 4)

        def body(c, carry):
            cur = idx_v[pl.ds(c * 16, 16)]
            nxt = idx_v[pl.ds((c + 1) * 16, 16)]
            for sub in range(4):
                g = c * 4 + sub
                if True:
                    fsub = sub + 2
                    if fsub < 4:
                        fire(g + 2, cur, 4 * fsub)
                    else:
                        fire(g + 2, nxt, 4 * (fsub - 4))
                process(g, cur, 4 * sub)
            return carry

        lax.fori_loop(0, n_chunks - 1, body, 0)
        clast = idx_v[pl.ds((n_chunks - 1) * 16, 16)]
        for sub in range(4):
            g = (n_chunks - 1) * 4 + sub
            if sub + 2 < 4:
                fire(g + 2, clast, 4 * (sub + 2))
            process(g, clast, 4 * sub)

        pltpu.sync_copy(rows_v, out_hbm.at[:, pl.ds(base, b_per_w)])

    return gather_k


def _mlp_body(xT_ref, eT_ref, w1dT_ref, w1eT_ref, b1_ref, w2T_ref, b2_ref, oT_ref):
    h = jnp.dot(w1dT_ref[...], xT_ref[...], preferred_element_type=jnp.float32)
    h = h + jnp.dot(w1eT_ref[...], eT_ref[...], preferred_element_type=jnp.float32)
    h = jnp.maximum(h + b1_ref[...], 0.0)
    o = jnp.dot(w2T_ref[...], h, preferred_element_type=jnp.float32) + b2_ref[...]
    oT_ref[...] = jnp.maximum(o, 0.0)


def _mlp(xT, embT, W1dT, W1eT, b1c, W2T, b2c, block_b=8192):
    F, B = xT.shape
    OUT, HID = W2T.shape
    D = embT.shape[0]
    grid = (B // block_b,)
    return pl.pallas_call(
        _mlp_body,
        grid=grid,
        in_specs=[
            pl.BlockSpec((F, block_b), lambda i: (0, i)),
            pl.BlockSpec((D, block_b), lambda i: (0, i)),
            pl.BlockSpec((HID, F), lambda i: (0, 0)),
            pl.BlockSpec((HID, D), lambda i: (0, 0)),
            pl.BlockSpec((HID, 1), lambda i: (0, 0)),
            pl.BlockSpec((OUT, HID), lambda i: (0, 0)),
            pl.BlockSpec((OUT, 1), lambda i: (0, 0)),
        ],
        out_specs=pl.BlockSpec((OUT, block_b), lambda i: (0, i)),
        out_shape=jax.ShapeDtypeStruct((OUT, B), jnp.float32),
    )(xT, embT, W1dT, W1eT, b1c, W2T, b2c)


def kernel(inputs, table, W1, b1, W2, b2):
    B, F = inputs.shape
    V, D = table.shape
    HID = W1.shape[1]
    tableT = table.T
    inputsT = inputs.T
    idx = inputsT[_IDX].astype(jnp.int32)
    embT = _make_sc_gather(V, D, B)(tableT, idx)
    W1T = W1.T
    W1dT = jnp.concatenate(
        [W1T[:, :_IDX], jnp.zeros((HID, 1), W1.dtype), W1T[:, _IDX : F - 1]],
        axis=1,
    )
    W1eT = W1T[:, F - 1 :]
    outT = _mlp(
        inputsT, embT, W1dT, W1eT, b1.reshape(-1, 1), W2.T, b2.reshape(-1, 1)
    )
    return outT.T
